# Initial kernel scaffold; baseline (speedup 1.0000x reference)
import jax
import jax.numpy as jnp
from jax.experimental import pallas as pl


def kernel(x, gate_w, W1, b1, W2, b2, W3, b3, Ws1, bs1, Ws2, bs2, Ws3, bs3):
    raise NotImplementedError("write your pallas kernel here")


# speedup vs baseline: 2.2484x; 2.2484x over previous
import functools

import jax
import jax.numpy as jnp
from jax import lax
from jax.experimental import pallas as pl
from jax.experimental.pallas import tpu as pltpu

D = 1024
F = 256
E = 64
T = 2048
FS = 2048
BT = 128
G = 96
PAD = G * BT
NPAIR = 2 * T
TRASH = NPAIR

_NEG = -1e30


def _gate_body(x_ref, gw_ref, ei0_ref, ei1_ref, ew0_ref, ew1_ref):
    xb = x_ref[...]
    l = lax.dot_general(xb, gw_ref[...], (((1,), (1,)), ((), ())),
                        preferred_element_type=jnp.float32)
    m0 = jnp.max(l, axis=1, keepdims=True)
    z = jnp.sum(jnp.exp(l - m0), axis=1, keepdims=True)
    ids = lax.broadcasted_iota(jnp.int32, l.shape, 1)
    a0 = jnp.min(jnp.where(l == m0, ids, E), axis=1, keepdims=True)
    lm = jnp.where(ids == a0, _NEG, l)
    m1 = jnp.max(lm, axis=1, keepdims=True)
    a1 = jnp.min(jnp.where(lm == m1, ids, E), axis=1, keepdims=True)
    p0 = 1.0 / z
    p1 = jnp.exp(m1 - m0) / z
    e0 = jnp.exp(p0)
    e1 = jnp.exp(p1)
    s = e0 + e1
    ei0_ref[...] = a0
    ei1_ref[...] = a1
    ew0_ref[...] = e0 / s
    ew1_ref[...] = e1 / s


def _gate(x2d, gate_w):
    TB = 256
    grid = (T // TB,)
    out = pl.pallas_call(
        _gate_body,
        grid=grid,
        in_specs=[
            pl.BlockSpec((TB, D), lambda i: (i, 0)),
            pl.BlockSpec((E, D), lambda i: (0, 0)),
        ],
        out_specs=[
            pl.BlockSpec((TB, 1), lambda i: (i, 0)),
            pl.BlockSpec((TB, 1), lambda i: (i, 0)),
            pl.BlockSpec((TB, 1), lambda i: (i, 0)),
            pl.BlockSpec((TB, 1), lambda i: (i, 0)),
        ],
        out_shape=[
            jax.ShapeDtypeStruct((T, 1), jnp.int32),
            jax.ShapeDtypeStruct((T, 1), jnp.int32),
            jax.ShapeDtypeStruct((T, 1), jnp.float32),
            jax.ShapeDtypeStruct((T, 1), jnp.float32),
        ],
    )(x2d, gate_w)
    return out


def _route_jnp(eflat, ewflat):
    counts = jnp.bincount(eflat, length=E)
    tcnt = (counts + BT - 1) // BT
    tile_start = jnp.cumsum(tcnt) - tcnt
    pad_off = tile_start * BT
    start = jnp.cumsum(counts) - counts
    order = jnp.argsort(eflat, stable=True)
    erank = jnp.arange(NPAIR) - start[eflat[order]]
    dst = pad_off[eflat[order]] + erank
    pw = jnp.zeros((PAD,), jnp.float32).at[dst].set(ewflat[order])
    stok = jnp.full((PAD,), TRASH, jnp.int32).at[dst].set(order.astype(jnp.int32))
    gidx = jnp.zeros((PAD,), jnp.int32).at[dst].set((order // 2).astype(jnp.int32))
    texp = jnp.clip(jnp.searchsorted(tile_start, jnp.arange(G), side="right") - 1,
                    0, E - 1).astype(jnp.int32)
    U = jnp.sum(tcnt)
    texp = jnp.where(jnp.arange(G) < U, texp, 0)
    return pw, stok, gidx, texp


def _ffn_body(texp_ref, xs_ref, w1_ref, b1_ref, w3_ref, b3_ref, w2_ref, b2_ref,
              pw_ref, yp_ref):
    xb = xs_ref[...]
    h1 = lax.dot_general(xb, w1_ref[0], (((1,), (1,)), ((), ())),
                         preferred_element_type=jnp.float32) + b1_ref[0]
    h3 = lax.dot_general(xb, w3_ref[0], (((1,), (1,)), ((), ())),
                         preferred_element_type=jnp.float32) + b3_ref[0]
    hp = h1 * h3
    h = hp * jax.nn.sigmoid(hp)
    o = lax.dot_general(h, w2_ref[0], (((1,), (1,)), ((), ())),
                        preferred_element_type=jnp.float32) + b2_ref[0]
    yp_ref[...] = o * pw_ref[...]


def _expert_ffn(texp, xs, W1, b1, W3, b3, W2, b2, pw2):
    b1r = b1.reshape(E, 1, F)
    b3r = b3.reshape(E, 1, F)
    b2r = b2.reshape(E, 1, D)
    grid_spec = pltpu.PrefetchScalarGridSpec(
        num_scalar_prefetch=1,
        grid=(G,),
        in_specs=[
            pl.BlockSpec((BT, D), lambda g, s: (g, 0)),
            pl.BlockSpec((1, F, D), lambda g, s: (s[g], 0, 0)),
            pl.BlockSpec((1, 1, F), lambda g, s: (s[g], 0, 0)),
            pl.BlockSpec((1, F, D), lambda g, s: (s[g], 0, 0)),
            pl.BlockSpec((1, 1, F), lambda g, s: (s[g], 0, 0)),
            pl.BlockSpec((1, D, F), lambda g, s: (s[g], 0, 0)),
            pl.BlockSpec((1, 1, D), lambda g, s: (s[g], 0, 0)),
            pl.BlockSpec((BT, 1), lambda g, s: (g, 0)),
        ],
        out_specs=pl.BlockSpec((BT, D), lambda g, s: (g, 0)),
    )
    return pl.pallas_call(
        _ffn_body,
        grid_spec=grid_spec,
        out_shape=jax.ShapeDtypeStruct((PAD, D), jnp.float32),
    )(texp, xs, W1, b1r, W3, b3r, W2, b2r, pw2)


def _shared_body(x_ref, ws1_ref, bs1_ref, ws3_ref, bs3_ref, ws2_ref, bs2_ref,
                 y01_ref, y_ref):
    xb = x_ref[...]
    h1 = lax.dot_general(xb, ws1_ref[...], (((1,), (1,)), ((), ())),
                         preferred_element_type=jnp.float32) + bs1_ref[...]
    h3 = lax.dot_general(xb, ws3_ref[...], (((1,), (1,)), ((), ())),
                         preferred_element_type=jnp.float32) + bs3_ref[...]
    hp = h1 * h3
    h = hp * jax.nn.sigmoid(hp)
    o = lax.dot_general(h, ws2_ref[...], (((1,), (1,)), ((), ())),
                        preferred_element_type=jnp.float32) + bs2_ref[...]
    yb = y01_ref[...]
    y_ref[...] = o + yb[:, :D] + yb[:, D:]


def _shared(x2d, Ws1, bs1, Ws3, bs3, Ws2, bs2, y01r):
    SB = 128
    grid = (T // SB,)
    return pl.pallas_call(
        _shared_body,
        grid=grid,
        in_specs=[
            pl.BlockSpec((SB, D), lambda i: (i, 0)),
            pl.BlockSpec((FS, D), lambda i: (0, 0)),
            pl.BlockSpec((1, FS), lambda i: (0, 0)),
            pl.BlockSpec((FS, D), lambda i: (0, 0)),
            pl.BlockSpec((1, FS), lambda i: (0, 0)),
            pl.BlockSpec((D, FS), lambda i: (0, 0)),
            pl.BlockSpec((1, D), lambda i: (0, 0)),
            pl.BlockSpec((SB, 2 * D), lambda i: (i, 0)),
        ],
        out_specs=pl.BlockSpec((SB, D), lambda i: (i, 0)),
        out_shape=jax.ShapeDtypeStruct((T, D), jnp.float32),
    )(x2d, Ws1, bs1.reshape(1, FS), Ws3, bs3.reshape(1, FS), Ws2,
      bs2.reshape(1, D), y01r)


def kernel(x, gate_w, W1, b1, W2, b2, W3, b3, Ws1, bs1, Ws2, bs2, Ws3, bs3):
    shape = x.shape
    x2d = x.reshape(T, D)

    ei0, ei1, ew0, ew1 = _gate(x2d, gate_w)
    eflat = jnp.concatenate([ei0, ei1], axis=1).reshape(-1)
    ewflat = jnp.concatenate([ew0, ew1], axis=1).reshape(-1)

    pw, stok, gidx, texp = _route_jnp(eflat, ewflat)

    xs = x2d[gidx]
    yp = _expert_ffn(texp, xs, W1, b1, W3, b3, W2, b2, pw.reshape(PAD, 1))

    y01 = jnp.zeros((NPAIR + 8, D), jnp.float32).at[stok].set(yp)
    y01r = y01[:NPAIR].reshape(T, 2 * D)

    y = _shared(x2d, Ws1, bs1, Ws3, bs3, Ws2, bs2, y01r)
    return y.reshape(shape)
